# Initial kernel scaffold; baseline (speedup 1.0000x reference)
import jax
import jax.numpy as jnp
from jax.experimental import pallas as pl


def kernel(xyz):
    raise NotImplementedError("write your pallas kernel here")


# speedup vs baseline: 11.6610x; 11.6610x over previous
import functools

import jax
import jax.numpy as jnp
from jax import lax
from jax.experimental import pallas as pl
from jax.experimental.pallas import tpu as pltpu

K = 16
R = 128


def _knn_block(q_ref, kt_ref, out_ref):
    q = q_ref[0]
    kt = kt_ref[0]
    cross = jnp.dot(q, kt, preferred_element_type=jnp.float32)
    q2 = jnp.sum(q * q, axis=1, keepdims=True)
    x2 = jnp.sum(kt * kt, axis=0, keepdims=True)
    d = (q2 + x2) - 2.0 * cross
    n = d.shape[1]
    col = lax.broadcasted_iota(jnp.int32, d.shape, 1)
    acc = jnp.zeros((d.shape[0], 128), jnp.int32)
    lane = lax.broadcasted_iota(jnp.int32, acc.shape, 1)
    for k in range(K):
        m = jnp.min(d, axis=1, keepdims=True)
        ji = jnp.min(jnp.where(d == m, col, jnp.int32(n)), axis=1,
                     keepdims=True)
        d = jnp.where(col == ji, jnp.float32(jnp.inf), d)
        acc = jnp.where(lane == k, ji, acc)
    out_ref[0] = acc[:, :K]


def kernel(xyz):
    b, n, _ = xyz.shape
    xyzp = jnp.pad(xyz, ((0, 0), (0, 0), (0, 5)))
    kt = xyzp.transpose(0, 2, 1)
    grid = (b, n // R)
    out = pl.pallas_call(
        _knn_block,
        grid=grid,
        in_specs=[
            pl.BlockSpec((1, R, 8), lambda i, j: (i, j, 0)),
            pl.BlockSpec((1, 8, n), lambda i, j: (i, 0, 0)),
        ],
        out_specs=pl.BlockSpec((1, R, K), lambda i, j: (i, j, 0)),
        out_shape=jax.ShapeDtypeStruct((b, n, K), jnp.int32),
    )(xyzp, kt)
    return out
